# Initial kernel scaffold; baseline (speedup 1.0000x reference)
import jax
import jax.numpy as jnp
from jax.experimental import pallas as pl


def kernel(features, edge_index1, edge_index2, W1, b1, Wc1, bc1, Wc2, bc2, W2, b2):
    raise NotImplementedError("write your pallas kernel here")


# speedup vs baseline: 1.6396x; 1.6396x over previous
import functools

import jax
import jax.numpy as jnp
from jax import lax
from jax.experimental import pallas as pl
from jax.experimental.pallas import tpu as pltpu
from jax.experimental.pallas import tpu_sc as plsc

N = 10000
E = 320000
D_IN = 128
H = 512
NB_GOS = 5000

NPAD = 10240
NCHUNK = 8
CW = 64
TILES = 16
CORES = 2
NSTEP = 160
BURST = 128
EPT = NSTEP * BURST
EPAD = EPT * TILES
ROWS_PER_TILE = NPAD // TILES
PASSES = NCHUNK // CORES


@functools.lru_cache(maxsize=None)
def _mesh():
    return plsc.VectorSubcoreMesh(
        core_axis_name="c", subcore_axis_name="s",
        num_cores=CORES, num_subcores=TILES)


def _zero_vmem_2d(ref, rows, cols):
    z16 = jnp.zeros((16,), jnp.float32)

    @pl.loop(0, rows)
    def _(r):
        for k in range(cols // 16):
            ref[r, pl.ds(k * 16, 16)] = z16


def _sc_degrees(edges):
    return pl.kernel(
        _sc_degrees_body,
        out_type=jax.ShapeDtypeStruct((4 * NPAD, 16), jnp.float32),
        mesh=_mesh(),
        scratch_types=[
            pltpu.VMEM((NSTEP, BURST), jnp.int32),
            pltpu.VMEM((BURST, 16), jnp.float32),
            pltpu.VMEM((64, 16), jnp.float32),
            pltpu.VMEM_SHARED((NPAD, 16), jnp.float32),
            pltpu.VMEM_SHARED((NPAD, 16), jnp.float32),
        ],
        compiler_params=pltpu.CompilerParams(use_tc_tiling_on_sc=False),
    )(edges)


def _sc_degrees_body(edges_hbm, out_hbm, idx_v, ones_v, zbuf, cnt0, cnt1):
    c = lax.axis_index("c")
    s = lax.axis_index("s")

    one16 = jnp.ones((16,), jnp.float32)

    @pl.loop(0, BURST)
    def _(r):
        ones_v[r, pl.ds(0, 16)] = one16

    _zero_vmem_2d(zbuf, 64, 16)

    for cnt in (cnt0, cnt1):
        for i in range(ROWS_PER_TILE // 64):
            pltpu.sync_copy(zbuf, cnt.at[pl.ds(s * ROWS_PER_TILE + i * 64, 64)])
    plsc.subcore_barrier()

    for w, cnt in ((0, cnt0), (1, cnt1)):
        fi = (c * 2 + w) * TILES + s
        pltpu.sync_copy(edges_hbm.at[fi], idx_v)

        @pl.loop(0, NSTEP)
        def _(j):
            pltpu.sync_copy(ones_v, cnt.at[idx_v.at[j]], add=True)

    plsc.subcore_barrier()

    for w, cnt in ((0, cnt0), (1, cnt1)):
        base = (c * 2 + w) * NPAD + s * ROWS_PER_TILE
        for i in range(ROWS_PER_TILE // BURST):
            pltpu.sync_copy(
                cnt.at[pl.ds(s * ROWS_PER_TILE + i * BURST, BURST)], ones_v)
            pltpu.sync_copy(ones_v, out_hbm.at[pl.ds(base + i * BURST, BURST)])


def _sc_conv(edges, ztab):
    return pl.kernel(
        _sc_conv_body,
        out_type=jax.ShapeDtypeStruct((NCHUNK * NPAD, CW), jnp.float32),
        mesh=_mesh(),
        scratch_types=[
            pltpu.VMEM((NSTEP, BURST), jnp.int32),
            pltpu.VMEM((NSTEP, BURST), jnp.int32),
            pltpu.VMEM((BURST, CW), jnp.float32),
            pltpu.VMEM((BURST, CW), jnp.float32),
            pltpu.VMEM((64, CW), jnp.float32),
            pltpu.VMEM_SHARED((NPAD, CW), jnp.float32),
            pltpu.SemaphoreType.DMA,
            pltpu.SemaphoreType.DMA,
        ],
        compiler_params=pltpu.CompilerParams(use_tc_tiling_on_sc=False),
    )(edges, ztab)


def _sc_conv_body(edges_hbm, ztab_hbm, out_hbm, src_v, dst_v, gb0, gb1, zbuf,
                  agg, sem0, sem1):
    c = lax.axis_index("c")
    s = lax.axis_index("s")

    _zero_vmem_2d(zbuf, 64, CW)

    pltpu.sync_copy(edges_hbm.at[s], src_v)
    pltpu.sync_copy(edges_hbm.at[TILES + s], dst_v)

    @pl.loop(0, NSTEP)
    def _(j):
        for k in range(BURST // 16):
            src_v[j, pl.ds(k * 16, 16)] = (
                src_v[j, pl.ds(k * 16, 16)] * NCHUNK + c)

    for p in range(PASSES):
        if p > 0:
            @pl.loop(0, NSTEP)
            def _(j):
                for k in range(BURST // 16):
                    src_v[j, pl.ds(k * 16, 16)] = (
                        src_v[j, pl.ds(k * 16, 16)] + CORES)

        for i in range(ROWS_PER_TILE // 64):
            pltpu.sync_copy(zbuf, agg.at[pl.ds(s * ROWS_PER_TILE + i * 64, 64)])
        plsc.subcore_barrier()

        @pl.loop(0, NSTEP)
        def _(j):
            pltpu.async_copy(ztab_hbm.at[src_v.at[j]], gb0, sem0).wait()
            pltpu.sync_copy(gb0, agg.at[dst_v.at[j]], add=True)

        plsc.subcore_barrier()

        cc = c + CORES * p
        base = cc * NPAD + s * ROWS_PER_TILE
        for i in range(ROWS_PER_TILE // BURST):
            pltpu.sync_copy(
                agg.at[pl.ds(s * ROWS_PER_TILE + i * BURST, BURST)], gb1)
            pltpu.sync_copy(gb1, out_hbm.at[pl.ds(base + i * BURST, BURST)])
        if p < PASSES - 1:
            plsc.subcore_barrier()


def _scale(cnt_blk):
    return lax.rsqrt(jnp.maximum(cnt_blk[:, :1], 1.0))


def _k1_body(f_ref, w_ref, b_ref, cnt_ref, o_ref):
    i = pl.program_id(0)
    x = jnp.maximum(
        jnp.dot(f_ref[...], w_ref[...], preferred_element_type=jnp.float32)
        + b_ref[0:1, :], 0.0)
    x = x * _scale(cnt_ref[...])
    rows = i * 1024 + lax.broadcasted_iota(jnp.int32, x.shape, 0)
    o_ref[...] = jnp.where(rows < N, x, 0.0)


def _tc_mlp(featp, W1, b1_8, cnt):
    return pl.pallas_call(
        _k1_body,
        grid=(NPAD // 1024,),
        in_specs=[
            pl.BlockSpec((1024, D_IN), lambda i: (i, 0)),
            pl.BlockSpec((D_IN, H), lambda i: (0, 0)),
            pl.BlockSpec((8, H), lambda i: (0, 0)),
            pl.BlockSpec((1024, 16), lambda i: (i, 0)),
        ],
        out_specs=pl.BlockSpec((1024, H), lambda i: (i, 0)),
        out_shape=jax.ShapeDtypeStruct((NPAD, H), jnp.float32),
    )(featp, W1, b1_8, cnt)


def _k3_body(a_ref, w_ref, b_ref, cin_ref, cout_ref, o_ref):
    ci = pl.program_id(1)
    i = pl.program_id(0)
    a = a_ref[0] * _scale(cin_ref[...])
    part = jnp.dot(a, w_ref[...], preferred_element_type=jnp.float32)

    @pl.when(ci == 0)
    def _():
        o_ref[...] = part

    @pl.when(ci > 0)
    def _():
        o_ref[...] = o_ref[...] + part

    @pl.when(ci == NCHUNK - 1)
    def _():
        h = (o_ref[...] + b_ref[0:1, :]) * _scale(cout_ref[...])
        rows = i * 1024 + lax.broadcasted_iota(jnp.int32, h.shape, 0)
        o_ref[...] = jnp.where(rows < N, h, 0.0)


def _tc_mid(agg1, Wc1, bc1_8, cnt_in, cnt_out):
    return pl.pallas_call(
        _k3_body,
        grid=(NPAD // 1024, NCHUNK),
        in_specs=[
            pl.BlockSpec((1, 1024, CW), lambda i, ci: (ci, i, 0)),
            pl.BlockSpec((CW, H), lambda i, ci: (ci, 0)),
            pl.BlockSpec((8, H), lambda i, ci: (0, 0)),
            pl.BlockSpec((1024, 16), lambda i, ci: (i, 0)),
            pl.BlockSpec((1024, 16), lambda i, ci: (i, 0)),
        ],
        out_specs=pl.BlockSpec((1024, H), lambda i, ci: (i, 0)),
        out_shape=jax.ShapeDtypeStruct((NPAD, H), jnp.float32),
    )(agg1, Wc1, bc1_8, cnt_in, cnt_out)


def _k0_body(wc_ref, w2_ref, bc_ref, b2_ref, wp_ref, bp_ref):
    w2 = w2_ref[...]
    wp_ref[...] = jnp.dot(wc_ref[...], w2, preferred_element_type=jnp.float32)
    bp_ref[...] = (
        jnp.dot(bc_ref[...], w2, preferred_element_type=jnp.float32)
        + b2_ref[...])


def _tc_fuse_w2(Wc2, W2, bc2_8, b2_8):
    return pl.pallas_call(
        _k0_body,
        grid=(1,),
        in_specs=[
            pl.BlockSpec((H, H), lambda j: (0, 0)),
            pl.BlockSpec((H, NB_GOS), lambda j: (0, 0)),
            pl.BlockSpec((8, H), lambda j: (0, 0)),
            pl.BlockSpec((8, NB_GOS), lambda j: (0, 0)),
        ],
        out_specs=[
            pl.BlockSpec((H, NB_GOS), lambda j: (0, 0)),
            pl.BlockSpec((8, NB_GOS), lambda j: (0, 0)),
        ],
        out_shape=[
            jax.ShapeDtypeStruct((H, NB_GOS), jnp.float32),
            jax.ShapeDtypeStruct((8, NB_GOS), jnp.float32),
        ],
    )(Wc2, W2, bc2_8, b2_8)


def _k4_body(a_ref, w_ref, b_ref, cnt_ref, o_ref):
    ci = pl.program_id(1)
    a = a_ref[0] * _scale(cnt_ref[...])
    part = jnp.dot(a, w_ref[...], preferred_element_type=jnp.float32)

    @pl.when(ci == 0)
    def _():
        o_ref[...] = part

    @pl.when(ci > 0)
    def _():
        o_ref[...] = o_ref[...] + part

    @pl.when(ci == NCHUNK - 1)
    def _():
        v = o_ref[...] + b_ref[0:1, :]
        o_ref[...] = 1.0 / (1.0 + jnp.exp(-v))


def _tc_out(agg2, W2p, b2p8, cnt_in):
    return pl.pallas_call(
        _k4_body,
        grid=(N // 400, NCHUNK),
        in_specs=[
            pl.BlockSpec((1, 400, CW), lambda i, ci: (ci, i, 0)),
            pl.BlockSpec((CW, NB_GOS), lambda i, ci: (ci, 0)),
            pl.BlockSpec((8, NB_GOS), lambda i, ci: (0, 0)),
            pl.BlockSpec((400, 16), lambda i, ci: (i, 0)),
        ],
        out_specs=pl.BlockSpec((400, NB_GOS), lambda i, ci: (i, 0)),
        out_shape=jax.ShapeDtypeStruct((N, NB_GOS), jnp.float32),
    )(agg2, W2p, b2p8, cnt_in)


def _prep_edges(ei):
    pad = jnp.full((EPAD - E,), N, dtype=jnp.int32)
    s = jnp.concatenate([ei[0], pad]).reshape(TILES, NSTEP, BURST)
    d = jnp.concatenate([ei[1], pad]).reshape(TILES, NSTEP, BURST)
    return s, d


def kernel(features, edge_index1, edge_index2, W1, b1, Wc1, bc1, Wc2, bc2,
           W2, b2):
    e1s, e1d = _prep_edges(edge_index1)
    e2s, e2d = _prep_edges(edge_index2)
    edges_deg = jnp.concatenate([e1s, e1d, e2s, e2d], axis=0)
    edges_g1 = jnp.concatenate([e1s, e1d], axis=0)
    edges_g2 = jnp.concatenate([e2s, e2d], axis=0)

    featp = jnp.pad(features, ((0, NPAD - N), (0, 0)))
    b1_8 = jnp.broadcast_to(b1.reshape(1, H), (8, H))
    bc1_8 = jnp.broadcast_to(bc1.reshape(1, H), (8, H))
    bc2_8 = jnp.broadcast_to(bc2.reshape(1, H), (8, H))
    b2_8 = jnp.broadcast_to(b2.reshape(1, NB_GOS), (8, NB_GOS))

    cnts = _sc_degrees(edges_deg).reshape(4, NPAD, 16)
    cnt_out1, cnt_in1, cnt_out2, cnt_in2 = (
        cnts[0], cnts[1], cnts[2], cnts[3])

    z1 = _tc_mlp(featp, W1, b1_8, cnt_out1)
    agg1 = _sc_conv(edges_g1, z1.reshape(NPAD * NCHUNK, CW))
    agg1 = agg1.reshape(NCHUNK, NPAD, CW)

    z2 = _tc_mid(agg1, Wc1, bc1_8, cnt_in1, cnt_out2)
    agg2 = _sc_conv(edges_g2, z2.reshape(NPAD * NCHUNK, CW))
    agg2 = agg2.reshape(NCHUNK, NPAD, CW)

    W2p, b2p8 = _tc_fuse_w2(Wc2, W2, bc2_8, b2_8)
    return _tc_out(agg2, W2p, b2p8, cnt_in2)


# speedup vs baseline: 1.7633x; 1.0754x over previous
import functools

import jax
import jax.numpy as jnp
from jax import lax
from jax.experimental import pallas as pl
from jax.experimental.pallas import tpu as pltpu
from jax.experimental.pallas import tpu_sc as plsc

N = 10000
E = 320000
D_IN = 128
H = 512
NB_GOS = 5000

NPAD = 10240
NCHUNK = 8
CW = 64
TILES = 16
CORES = 2
NSTEP = 160
BURST = 128
EPT = NSTEP * BURST
EPAD = EPT * TILES
ROWS_PER_TILE = NPAD // TILES
PASSES = NCHUNK // CORES


@functools.lru_cache(maxsize=None)
def _mesh():
    return plsc.VectorSubcoreMesh(
        core_axis_name="c", subcore_axis_name="s",
        num_cores=CORES, num_subcores=TILES)


def _zero_vmem_2d(ref, rows, cols):
    z16 = jnp.zeros((16,), jnp.float32)

    @pl.loop(0, rows)
    def _(r):
        for k in range(cols // 16):
            ref[r, pl.ds(k * 16, 16)] = z16


def _sc_degrees(edges):
    return pl.kernel(
        _sc_degrees_body,
        out_type=jax.ShapeDtypeStruct((4 * NPAD, 16), jnp.float32),
        mesh=_mesh(),
        scratch_types=[
            pltpu.VMEM((NSTEP, BURST), jnp.int32),
            pltpu.VMEM((BURST, 16), jnp.float32),
            pltpu.VMEM((64, 16), jnp.float32),
            pltpu.VMEM_SHARED((NPAD, 16), jnp.float32),
            pltpu.VMEM_SHARED((NPAD, 16), jnp.float32),
        ],
        compiler_params=pltpu.CompilerParams(use_tc_tiling_on_sc=False),
    )(edges)


def _sc_degrees_body(edges_hbm, out_hbm, idx_v, ones_v, zbuf, cnt0, cnt1):
    c = lax.axis_index("c")
    s = lax.axis_index("s")

    one16 = jnp.ones((16,), jnp.float32)

    @pl.loop(0, BURST)
    def _(r):
        ones_v[r, pl.ds(0, 16)] = one16

    _zero_vmem_2d(zbuf, 64, 16)

    for cnt in (cnt0, cnt1):
        for i in range(ROWS_PER_TILE // 64):
            pltpu.sync_copy(zbuf, cnt.at[pl.ds(s * ROWS_PER_TILE + i * 64, 64)])
    plsc.subcore_barrier()

    for w, cnt in ((0, cnt0), (1, cnt1)):
        fi = (c * 2 + w) * TILES + s
        pltpu.sync_copy(edges_hbm.at[fi], idx_v)

        @pl.loop(0, NSTEP)
        def _(j):
            pltpu.sync_copy(ones_v, cnt.at[idx_v.at[j]], add=True)

    plsc.subcore_barrier()

    for w, cnt in ((0, cnt0), (1, cnt1)):
        base = (c * 2 + w) * NPAD + s * ROWS_PER_TILE
        for i in range(ROWS_PER_TILE // BURST):
            pltpu.sync_copy(
                cnt.at[pl.ds(s * ROWS_PER_TILE + i * BURST, BURST)], ones_v)
            pltpu.sync_copy(ones_v, out_hbm.at[pl.ds(base + i * BURST, BURST)])


def _sc_conv(edges, ztab):
    return pl.kernel(
        _sc_conv_body,
        out_type=jax.ShapeDtypeStruct((NCHUNK * NPAD, CW), jnp.float32),
        mesh=_mesh(),
        scratch_types=[
            pltpu.VMEM((NSTEP, BURST), jnp.int32),
            pltpu.VMEM((NSTEP, BURST), jnp.int32),
            pltpu.VMEM((BURST, CW), jnp.float32),
            pltpu.VMEM((BURST, CW), jnp.float32),
            pltpu.VMEM((64, CW), jnp.float32),
            pltpu.VMEM_SHARED((NPAD, CW), jnp.float32),
            pltpu.SemaphoreType.DMA,
            pltpu.SemaphoreType.DMA,
            pltpu.SemaphoreType.DMA,
            pltpu.SemaphoreType.DMA,
        ],
        compiler_params=pltpu.CompilerParams(use_tc_tiling_on_sc=False),
    )(edges, ztab)


def _sc_conv_body(edges_hbm, ztab_hbm, out_hbm, src_v, dst_v, gb0, gb1, zbuf,
                  agg, sem0, sem1, ssem0, ssem1):
    c = lax.axis_index("c")
    s = lax.axis_index("s")

    _zero_vmem_2d(zbuf, 64, CW)

    pltpu.sync_copy(edges_hbm.at[s], src_v)
    pltpu.sync_copy(edges_hbm.at[TILES + s], dst_v)

    @pl.loop(0, NSTEP)
    def _(j):
        for k in range(BURST // 16):
            src_v[j, pl.ds(k * 16, 16)] = (
                src_v[j, pl.ds(k * 16, 16)] * NCHUNK + c)

    for p in range(PASSES):
        if p > 0:
            @pl.loop(0, NSTEP)
            def _(j):
                for k in range(BURST // 16):
                    src_v[j, pl.ds(k * 16, 16)] = (
                        src_v[j, pl.ds(k * 16, 16)] + CORES)

        for i in range(ROWS_PER_TILE // 64):
            pltpu.sync_copy(zbuf, agg.at[pl.ds(s * ROWS_PER_TILE + i * 64, 64)])
        plsc.subcore_barrier()

        pltpu.async_copy(ztab_hbm.at[src_v.at[0]], gb0, sem0)
        pltpu.async_copy(ztab_hbm.at[src_v.at[1]], gb1, sem1)

        def g_wait(sem, buf):
            pltpu.make_async_copy(ztab_hbm.at[src_v.at[0]], buf, sem).wait()

        def s_wait(sem, buf):
            pltpu.make_async_copy(buf, agg.at[dst_v.at[0]], sem).wait()

        @pl.loop(0, NSTEP // 2)
        def _(jj):
            j = jj * 2
            g_wait(sem0, gb0)
            pltpu.async_copy(gb0, agg.at[dst_v.at[j]], ssem0, add=True)
            g_wait(sem1, gb1)
            pltpu.async_copy(gb1, agg.at[dst_v.at[j + 1]], ssem1, add=True)
            s_wait(ssem0, gb0)

            @pl.when(j + 2 < NSTEP)
            def _():
                pltpu.async_copy(ztab_hbm.at[src_v.at[j + 2]], gb0, sem0)

            s_wait(ssem1, gb1)

            @pl.when(j + 3 < NSTEP)
            def _():
                pltpu.async_copy(ztab_hbm.at[src_v.at[j + 3]], gb1, sem1)

        plsc.subcore_barrier()

        cc = c + CORES * p
        base = cc * NPAD + s * ROWS_PER_TILE
        for i in range(ROWS_PER_TILE // BURST):
            pltpu.sync_copy(
                agg.at[pl.ds(s * ROWS_PER_TILE + i * BURST, BURST)], gb1)
            pltpu.sync_copy(gb1, out_hbm.at[pl.ds(base + i * BURST, BURST)])
        if p < PASSES - 1:
            plsc.subcore_barrier()


def _scale(cnt_blk):
    return lax.rsqrt(jnp.maximum(cnt_blk[:, :1], 1.0))


def _k1_body(f_ref, w_ref, b_ref, cnt_ref, o_ref):
    i = pl.program_id(0)
    x = jnp.maximum(
        jnp.dot(f_ref[...], w_ref[...], preferred_element_type=jnp.float32)
        + b_ref[0:1, :], 0.0)
    x = x * _scale(cnt_ref[...])
    rows = i * 1024 + lax.broadcasted_iota(jnp.int32, x.shape, 0)
    o_ref[...] = jnp.where(rows < N, x, 0.0)


def _tc_mlp(featp, W1, b1_8, cnt):
    return pl.pallas_call(
        _k1_body,
        grid=(NPAD // 1024,),
        in_specs=[
            pl.BlockSpec((1024, D_IN), lambda i: (i, 0)),
            pl.BlockSpec((D_IN, H), lambda i: (0, 0)),
            pl.BlockSpec((8, H), lambda i: (0, 0)),
            pl.BlockSpec((1024, 16), lambda i: (i, 0)),
        ],
        out_specs=pl.BlockSpec((1024, H), lambda i: (i, 0)),
        out_shape=jax.ShapeDtypeStruct((NPAD, H), jnp.float32),
    )(featp, W1, b1_8, cnt)


def _k3_body(a_ref, w_ref, b_ref, cin_ref, cout_ref, o_ref):
    ci = pl.program_id(1)
    i = pl.program_id(0)
    a = a_ref[0] * _scale(cin_ref[...])
    part = jnp.dot(a, w_ref[...], preferred_element_type=jnp.float32)

    @pl.when(ci == 0)
    def _():
        o_ref[...] = part

    @pl.when(ci > 0)
    def _():
        o_ref[...] = o_ref[...] + part

    @pl.when(ci == NCHUNK - 1)
    def _():
        h = (o_ref[...] + b_ref[0:1, :]) * _scale(cout_ref[...])
        rows = i * 1024 + lax.broadcasted_iota(jnp.int32, h.shape, 0)
        o_ref[...] = jnp.where(rows < N, h, 0.0)


def _tc_mid(agg1, Wc1, bc1_8, cnt_in, cnt_out):
    return pl.pallas_call(
        _k3_body,
        grid=(NPAD // 1024, NCHUNK),
        in_specs=[
            pl.BlockSpec((1, 1024, CW), lambda i, ci: (ci, i, 0)),
            pl.BlockSpec((CW, H), lambda i, ci: (ci, 0)),
            pl.BlockSpec((8, H), lambda i, ci: (0, 0)),
            pl.BlockSpec((1024, 16), lambda i, ci: (i, 0)),
            pl.BlockSpec((1024, 16), lambda i, ci: (i, 0)),
        ],
        out_specs=pl.BlockSpec((1024, H), lambda i, ci: (i, 0)),
        out_shape=jax.ShapeDtypeStruct((NPAD, H), jnp.float32),
    )(agg1, Wc1, bc1_8, cnt_in, cnt_out)


def _k0_body(wc_ref, w2_ref, bc_ref, b2_ref, wp_ref, bp_ref):
    w2 = w2_ref[...]
    wp_ref[...] = jnp.dot(wc_ref[...], w2, preferred_element_type=jnp.float32)
    bp_ref[...] = (
        jnp.dot(bc_ref[...], w2, preferred_element_type=jnp.float32)
        + b2_ref[...])


def _tc_fuse_w2(Wc2, W2, bc2_8, b2_8):
    return pl.pallas_call(
        _k0_body,
        grid=(1,),
        in_specs=[
            pl.BlockSpec((H, H), lambda j: (0, 0)),
            pl.BlockSpec((H, NB_GOS), lambda j: (0, 0)),
            pl.BlockSpec((8, H), lambda j: (0, 0)),
            pl.BlockSpec((8, NB_GOS), lambda j: (0, 0)),
        ],
        out_specs=[
            pl.BlockSpec((H, NB_GOS), lambda j: (0, 0)),
            pl.BlockSpec((8, NB_GOS), lambda j: (0, 0)),
        ],
        out_shape=[
            jax.ShapeDtypeStruct((H, NB_GOS), jnp.float32),
            jax.ShapeDtypeStruct((8, NB_GOS), jnp.float32),
        ],
    )(Wc2, W2, bc2_8, b2_8)


def _k4_body(a_ref, w_ref, b_ref, cnt_ref, o_ref):
    ci = pl.program_id(1)
    a = a_ref[0] * _scale(cnt_ref[...])
    part = jnp.dot(a, w_ref[...], preferred_element_type=jnp.float32)

    @pl.when(ci == 0)
    def _():
        o_ref[...] = part

    @pl.when(ci > 0)
    def _():
        o_ref[...] = o_ref[...] + part

    @pl.when(ci == NCHUNK - 1)
    def _():
        v = o_ref[...] + b_ref[0:1, :]
        o_ref[...] = 1.0 / (1.0 + jnp.exp(-v))


def _tc_out(agg2, W2p, b2p8, cnt_in):
    return pl.pallas_call(
        _k4_body,
        grid=(N // 400, NCHUNK),
        in_specs=[
            pl.BlockSpec((1, 400, CW), lambda i, ci: (ci, i, 0)),
            pl.BlockSpec((CW, NB_GOS), lambda i, ci: (ci, 0)),
            pl.BlockSpec((8, NB_GOS), lambda i, ci: (0, 0)),
            pl.BlockSpec((400, 16), lambda i, ci: (i, 0)),
        ],
        out_specs=pl.BlockSpec((400, NB_GOS), lambda i, ci: (i, 0)),
        out_shape=jax.ShapeDtypeStruct((N, NB_GOS), jnp.float32),
    )(agg2, W2p, b2p8, cnt_in)


def _prep_edges(ei):
    pad = jnp.full((EPAD - E,), N, dtype=jnp.int32)
    s = jnp.concatenate([ei[0], pad]).reshape(TILES, NSTEP, BURST)
    d = jnp.concatenate([ei[1], pad]).reshape(TILES, NSTEP, BURST)
    return s, d


def kernel(features, edge_index1, edge_index2, W1, b1, Wc1, bc1, Wc2, bc2,
           W2, b2):
    e1s, e1d = _prep_edges(edge_index1)
    e2s, e2d = _prep_edges(edge_index2)
    edges_deg = jnp.concatenate([e1s, e1d, e2s, e2d], axis=0)
    edges_g1 = jnp.concatenate([e1s, e1d], axis=0)
    edges_g2 = jnp.concatenate([e2s, e2d], axis=0)

    featp = jnp.pad(features, ((0, NPAD - N), (0, 0)))
    b1_8 = jnp.broadcast_to(b1.reshape(1, H), (8, H))
    bc1_8 = jnp.broadcast_to(bc1.reshape(1, H), (8, H))
    bc2_8 = jnp.broadcast_to(bc2.reshape(1, H), (8, H))
    b2_8 = jnp.broadcast_to(b2.reshape(1, NB_GOS), (8, NB_GOS))

    cnts = _sc_degrees(edges_deg).reshape(4, NPAD, 16)
    cnt_out1, cnt_in1, cnt_out2, cnt_in2 = (
        cnts[0], cnts[1], cnts[2], cnts[3])

    z1 = _tc_mlp(featp, W1, b1_8, cnt_out1)
    agg1 = _sc_conv(edges_g1, z1.reshape(NPAD * NCHUNK, CW))
    agg1 = agg1.reshape(NCHUNK, NPAD, CW)

    z2 = _tc_mid(agg1, Wc1, bc1_8, cnt_in1, cnt_out2)
    agg2 = _sc_conv(edges_g2, z2.reshape(NPAD * NCHUNK, CW))
    agg2 = agg2.reshape(NCHUNK, NPAD, CW)

    W2p, b2p8 = _tc_fuse_w2(Wc2, W2, bc2_8, b2_8)
    return _tc_out(agg2, W2p, b2p8, cnt_in2)


# speedup vs baseline: 1.8750x; 1.0633x over previous
import functools

import jax
import jax.numpy as jnp
from jax import lax
from jax.experimental import pallas as pl
from jax.experimental.pallas import tpu as pltpu
from jax.experimental.pallas import tpu_sc as plsc

N = 10000
E = 320000
D_IN = 128
H = 512
NB_GOS = 5000

NPAD = 10240
NCHUNK = 8
CW = 64
TILES = 16
CORES = 2
NSTEP = 160
BURST = 128
EPT = NSTEP * BURST
EPAD = EPT * TILES
ROWS_PER_TILE = NPAD // TILES
PASSES = NCHUNK // CORES


@functools.lru_cache(maxsize=None)
def _mesh():
    return plsc.VectorSubcoreMesh(
        core_axis_name="c", subcore_axis_name="s",
        num_cores=CORES, num_subcores=TILES)


def _zero_vmem_2d(ref, rows, cols):
    z16 = jnp.zeros((16,), jnp.float32)

    @pl.loop(0, rows)
    def _(r):
        for k in range(cols // 16):
            ref[r, pl.ds(k * 16, 16)] = z16


def _sc_degrees(edges):
    return pl.kernel(
        _sc_degrees_body,
        out_type=jax.ShapeDtypeStruct((4 * NPAD, 16), jnp.float32),
        mesh=_mesh(),
        scratch_types=[
            pltpu.VMEM((NSTEP, BURST), jnp.int32),
            pltpu.VMEM((BURST, 16), jnp.float32),
            pltpu.VMEM((64, 16), jnp.float32),
            pltpu.VMEM_SHARED((NPAD, 16), jnp.float32),
            pltpu.VMEM_SHARED((NPAD, 16), jnp.float32),
        ],
        compiler_params=pltpu.CompilerParams(use_tc_tiling_on_sc=False),
    )(edges)


def _sc_degrees_body(edges_hbm, out_hbm, idx_v, ones_v, zbuf, cnt0, cnt1):
    c = lax.axis_index("c")
    s = lax.axis_index("s")

    one16 = jnp.ones((16,), jnp.float32)

    @pl.loop(0, BURST)
    def _(r):
        ones_v[r, pl.ds(0, 16)] = one16

    _zero_vmem_2d(zbuf, 64, 16)

    for cnt in (cnt0, cnt1):
        for i in range(ROWS_PER_TILE // 64):
            pltpu.sync_copy(zbuf, cnt.at[pl.ds(s * ROWS_PER_TILE + i * 64, 64)])
    plsc.subcore_barrier()

    for w, cnt in ((0, cnt0), (1, cnt1)):
        fi = (c * 2 + w) * TILES + s
        pltpu.sync_copy(edges_hbm.at[fi], idx_v)

        @pl.loop(0, NSTEP)
        def _(j):
            pltpu.sync_copy(ones_v, cnt.at[idx_v.at[j]], add=True)

    plsc.subcore_barrier()

    for w, cnt in ((0, cnt0), (1, cnt1)):
        base = (c * 2 + w) * NPAD + s * ROWS_PER_TILE
        for i in range(ROWS_PER_TILE // BURST):
            pltpu.sync_copy(
                cnt.at[pl.ds(s * ROWS_PER_TILE + i * BURST, BURST)], ones_v)
            pltpu.sync_copy(ones_v, out_hbm.at[pl.ds(base + i * BURST, BURST)])


def _sc_conv(edges, ztab):
    return pl.kernel(
        _sc_conv_body,
        out_type=jax.ShapeDtypeStruct((NCHUNK * NPAD, CW), jnp.float32),
        mesh=_mesh(),
        scratch_types=[
            pltpu.VMEM((NSTEP, BURST), jnp.int32),
            pltpu.VMEM((NSTEP, BURST), jnp.int32),
            pltpu.VMEM((BURST, CW), jnp.float32),
            pltpu.VMEM((BURST, CW), jnp.float32),
            pltpu.VMEM((BURST, CW), jnp.float32),
            pltpu.VMEM((BURST, CW), jnp.float32),
            pltpu.VMEM((64, CW), jnp.float32),
            pltpu.VMEM_SHARED((NPAD, CW), jnp.float32),
            pltpu.SemaphoreType.DMA,
            pltpu.SemaphoreType.DMA,
            pltpu.SemaphoreType.DMA,
            pltpu.SemaphoreType.DMA,
            pltpu.SemaphoreType.DMA,
            pltpu.SemaphoreType.DMA,
            pltpu.SemaphoreType.DMA,
            pltpu.SemaphoreType.DMA,
        ],
        compiler_params=pltpu.CompilerParams(use_tc_tiling_on_sc=False),
    )(edges, ztab)


def _sc_conv_body(edges_hbm, ztab_hbm, out_hbm, src_v, dst_v, gb0, gb1, gb2,
                  gb3, zbuf, agg, sem0, sem1, sem2, sem3,
                  ssem0, ssem1, ssem2, ssem3):
    c = lax.axis_index("c")
    s = lax.axis_index("s")

    _zero_vmem_2d(zbuf, 64, CW)

    pltpu.sync_copy(edges_hbm.at[s], src_v)
    pltpu.sync_copy(edges_hbm.at[TILES + s], dst_v)

    @pl.loop(0, NSTEP)
    def _(j):
        for k in range(BURST // 16):
            src_v[j, pl.ds(k * 16, 16)] = (
                src_v[j, pl.ds(k * 16, 16)] * NCHUNK + c)

    for p in range(PASSES):
        if p > 0:
            @pl.loop(0, NSTEP)
            def _(j):
                for k in range(BURST // 16):
                    src_v[j, pl.ds(k * 16, 16)] = (
                        src_v[j, pl.ds(k * 16, 16)] + CORES)

        for i in range(ROWS_PER_TILE // 64):
            pltpu.sync_copy(zbuf, agg.at[pl.ds(s * ROWS_PER_TILE + i * 64, 64)])
        plsc.subcore_barrier()

        gbufs = (gb0, gb1, gb2, gb3)
        gsems = (sem0, sem1, sem2, sem3)
        ssems = (ssem0, ssem1, ssem2, ssem3)

        def g_wait(sem, buf):
            pltpu.make_async_copy(ztab_hbm.at[src_v.at[0]], buf, sem).wait()

        def s_wait(sem, buf):
            pltpu.make_async_copy(buf, agg.at[dst_v.at[0]], sem).wait()

        for b in range(4):
            pltpu.async_copy(ztab_hbm.at[src_v.at[b]], gbufs[b], gsems[b])

        @pl.loop(0, NSTEP // 4)
        def _(jj):
            j = jj * 4
            for b in range(4):
                g_wait(gsems[b], gbufs[b])
                pltpu.async_copy(
                    gbufs[b], agg.at[dst_v.at[j + b]], ssems[b], add=True)
            for b in range(4):
                s_wait(ssems[b], gbufs[b])

                @pl.when(j + 4 + b < NSTEP)
                def _():
                    pltpu.async_copy(
                        ztab_hbm.at[src_v.at[j + 4 + b]], gbufs[b], gsems[b])

        plsc.subcore_barrier()

        cc = c + CORES * p
        base = cc * NPAD + s * ROWS_PER_TILE
        for i in range(ROWS_PER_TILE // BURST):
            pltpu.sync_copy(
                agg.at[pl.ds(s * ROWS_PER_TILE + i * BURST, BURST)], gb1)
            pltpu.sync_copy(gb1, out_hbm.at[pl.ds(base + i * BURST, BURST)])
        if p < PASSES - 1:
            plsc.subcore_barrier()


def _scale(cnt_blk):
    return lax.rsqrt(jnp.maximum(cnt_blk[:, :1], 1.0))


def _k1_body(f_ref, w_ref, b_ref, cnt_ref, o_ref):
    i = pl.program_id(0)
    x = jnp.maximum(
        jnp.dot(f_ref[...], w_ref[...], preferred_element_type=jnp.float32)
        + b_ref[0:1, :], 0.0)
    x = x * _scale(cnt_ref[...])
    rows = i * 1024 + lax.broadcasted_iota(jnp.int32, x.shape, 0)
    o_ref[...] = jnp.where(rows < N, x, 0.0)


def _tc_mlp(featp, W1, b1_8, cnt):
    return pl.pallas_call(
        _k1_body,
        grid=(NPAD // 1024,),
        in_specs=[
            pl.BlockSpec((1024, D_IN), lambda i: (i, 0)),
            pl.BlockSpec((D_IN, H), lambda i: (0, 0)),
            pl.BlockSpec((8, H), lambda i: (0, 0)),
            pl.BlockSpec((1024, 16), lambda i: (i, 0)),
        ],
        out_specs=pl.BlockSpec((1024, H), lambda i: (i, 0)),
        out_shape=jax.ShapeDtypeStruct((NPAD, H), jnp.float32),
    )(featp, W1, b1_8, cnt)


def _k3_body(a_ref, w_ref, b_ref, cin_ref, cout_ref, o_ref):
    ci = pl.program_id(1)
    i = pl.program_id(0)
    a = a_ref[0] * _scale(cin_ref[...])
    part = jnp.dot(a, w_ref[...], preferred_element_type=jnp.float32)

    @pl.when(ci == 0)
    def _():
        o_ref[...] = part

    @pl.when(ci > 0)
    def _():
        o_ref[...] = o_ref[...] + part

    @pl.when(ci == NCHUNK - 1)
    def _():
        h = (o_ref[...] + b_ref[0:1, :]) * _scale(cout_ref[...])
        rows = i * 1024 + lax.broadcasted_iota(jnp.int32, h.shape, 0)
        o_ref[...] = jnp.where(rows < N, h, 0.0)


def _tc_mid(agg1, Wc1, bc1_8, cnt_in, cnt_out):
    return pl.pallas_call(
        _k3_body,
        grid=(NPAD // 1024, NCHUNK),
        in_specs=[
            pl.BlockSpec((1, 1024, CW), lambda i, ci: (ci, i, 0)),
            pl.BlockSpec((CW, H), lambda i, ci: (ci, 0)),
            pl.BlockSpec((8, H), lambda i, ci: (0, 0)),
            pl.BlockSpec((1024, 16), lambda i, ci: (i, 0)),
            pl.BlockSpec((1024, 16), lambda i, ci: (i, 0)),
        ],
        out_specs=pl.BlockSpec((1024, H), lambda i, ci: (i, 0)),
        out_shape=jax.ShapeDtypeStruct((NPAD, H), jnp.float32),
    )(agg1, Wc1, bc1_8, cnt_in, cnt_out)


def _k0_body(wc_ref, w2_ref, bc_ref, b2_ref, wp_ref, bp_ref):
    w2 = w2_ref[...]
    wp_ref[...] = jnp.dot(wc_ref[...], w2, preferred_element_type=jnp.float32)
    bp_ref[...] = (
        jnp.dot(bc_ref[...], w2, preferred_element_type=jnp.float32)
        + b2_ref[...])


def _tc_fuse_w2(Wc2, W2, bc2_8, b2_8):
    return pl.pallas_call(
        _k0_body,
        grid=(1,),
        in_specs=[
            pl.BlockSpec((H, H), lambda j: (0, 0)),
            pl.BlockSpec((H, NB_GOS), lambda j: (0, 0)),
            pl.BlockSpec((8, H), lambda j: (0, 0)),
            pl.BlockSpec((8, NB_GOS), lambda j: (0, 0)),
        ],
        out_specs=[
            pl.BlockSpec((H, NB_GOS), lambda j: (0, 0)),
            pl.BlockSpec((8, NB_GOS), lambda j: (0, 0)),
        ],
        out_shape=[
            jax.ShapeDtypeStruct((H, NB_GOS), jnp.float32),
            jax.ShapeDtypeStruct((8, NB_GOS), jnp.float32),
        ],
    )(Wc2, W2, bc2_8, b2_8)


def _k4_body(a_ref, w_ref, b_ref, cnt_ref, o_ref):
    ci = pl.program_id(1)
    a = a_ref[0] * _scale(cnt_ref[...])
    part = jnp.dot(a, w_ref[...], preferred_element_type=jnp.float32)

    @pl.when(ci == 0)
    def _():
        o_ref[...] = part

    @pl.when(ci > 0)
    def _():
        o_ref[...] = o_ref[...] + part

    @pl.when(ci == NCHUNK - 1)
    def _():
        v = o_ref[...] + b_ref[0:1, :]
        o_ref[...] = 1.0 / (1.0 + jnp.exp(-v))


def _tc_out(agg2, W2p, b2p8, cnt_in):
    return pl.pallas_call(
        _k4_body,
        grid=(N // 400, NCHUNK),
        in_specs=[
            pl.BlockSpec((1, 400, CW), lambda i, ci: (ci, i, 0)),
            pl.BlockSpec((CW, NB_GOS), lambda i, ci: (ci, 0)),
            pl.BlockSpec((8, NB_GOS), lambda i, ci: (0, 0)),
            pl.BlockSpec((400, 16), lambda i, ci: (i, 0)),
        ],
        out_specs=pl.BlockSpec((400, NB_GOS), lambda i, ci: (i, 0)),
        out_shape=jax.ShapeDtypeStruct((N, NB_GOS), jnp.float32),
    )(agg2, W2p, b2p8, cnt_in)


def _prep_edges(ei):
    pad = jnp.full((EPAD - E,), N, dtype=jnp.int32)
    s = jnp.concatenate([ei[0], pad]).reshape(TILES, NSTEP, BURST)
    d = jnp.concatenate([ei[1], pad]).reshape(TILES, NSTEP, BURST)
    return s, d


def kernel(features, edge_index1, edge_index2, W1, b1, Wc1, bc1, Wc2, bc2,
           W2, b2):
    e1s, e1d = _prep_edges(edge_index1)
    e2s, e2d = _prep_edges(edge_index2)
    edges_deg = jnp.concatenate([e1s, e1d, e2s, e2d], axis=0)
    edges_g1 = jnp.concatenate([e1s, e1d], axis=0)
    edges_g2 = jnp.concatenate([e2s, e2d], axis=0)

    featp = jnp.pad(features, ((0, NPAD - N), (0, 0)))
    b1_8 = jnp.broadcast_to(b1.reshape(1, H), (8, H))
    bc1_8 = jnp.broadcast_to(bc1.reshape(1, H), (8, H))
    bc2_8 = jnp.broadcast_to(bc2.reshape(1, H), (8, H))
    b2_8 = jnp.broadcast_to(b2.reshape(1, NB_GOS), (8, NB_GOS))

    cnts = _sc_degrees(edges_deg).reshape(4, NPAD, 16)
    cnt_out1, cnt_in1, cnt_out2, cnt_in2 = (
        cnts[0], cnts[1], cnts[2], cnts[3])

    z1 = _tc_mlp(featp, W1, b1_8, cnt_out1)
    agg1 = _sc_conv(edges_g1, z1.reshape(NPAD * NCHUNK, CW))
    agg1 = agg1.reshape(NCHUNK, NPAD, CW)

    z2 = _tc_mid(agg1, Wc1, bc1_8, cnt_in1, cnt_out2)
    agg2 = _sc_conv(edges_g2, z2.reshape(NPAD * NCHUNK, CW))
    agg2 = agg2.reshape(NCHUNK, NPAD, CW)

    W2p, b2p8 = _tc_fuse_w2(Wc2, W2, bc2_8, b2_8)
    return _tc_out(agg2, W2p, b2p8, cnt_in2)
